# baseline (speedup 1.0000x reference)
import functools

import jax
import jax.numpy as jnp
from jax import lax
from jax.experimental import pallas as pl
from jax.experimental.pallas import tpu as pltpu
from jax.experimental.pallas import tpu_sc as plsc

_B, _T, _D = 64, 576, 64
_N = _B * _T
_K = 1024

_R = 1024
_STEPS = _N // _R

_NC, _NS = 2, 16
_NW = _NC * _NS
_ROWS_PER_TILE = _N // _NW
_CHUNK = 128
_NCHUNK = _ROWS_PER_TILE // _CHUNK


def _tc_body(z_ref, cb_ref, cbn_ref, ids_ref, idx_ref, loss_ref):
    i = pl.program_id(0)
    zb = z_ref[...]

    @pl.when(i == 0)
    def _init():
        loss_ref[0, 0] = 0.0

    dots = lax.dot_general(zb, cb_ref[...], (((1,), (1,)), ((), ())),
                           preferred_element_type=jnp.float32)
    rn = jnp.sum(zb * zb, axis=1, keepdims=True)
    d2 = rn - 2.0 * dots + cbn_ref[...]
    v = d2[:, 0:128]
    bi = jnp.zeros((_R, 128), jnp.float32)
    for c in range(1, 8):
        vc = d2[:, c * 128:(c + 1) * 128]
        take = vc < v
        v = jnp.where(take, vc, v)
        bi = jnp.where(take, float(c), bi)
    m = jnp.min(v, axis=1)
    lane = ids_ref[0, 0:128][None, :]
    cand = bi * 128.0 + jnp.broadcast_to(lane, (_R, 128))
    idxf = jnp.min(jnp.where(v == m[:, None], cand, float(_K)), axis=1)
    idx_ref[...] = idxf.astype(jnp.int32)
    loss_ref[0, 0] += jnp.sum(m)

    @pl.when(i == _STEPS - 1)
    def _finish():
        loss_ref[0, 0] = loss_ref[0, 0] * (1.25 / (_N * _D))


_tc_call = pl.pallas_call(
    _tc_body,
    grid=(_STEPS,),
    in_specs=[
        pl.BlockSpec((_R, _D), lambda i: (i, 0)),
        pl.BlockSpec((_K, _D), lambda i: (0, 0)),
        pl.BlockSpec((1, _K), lambda i: (0, 0)),
        pl.BlockSpec((1, _K), lambda i: (0, 0)),
    ],
    out_specs=[
        pl.BlockSpec((_R,), lambda i: (i,)),
        pl.BlockSpec(memory_space=pltpu.SMEM, block_shape=(1, 1),
                     index_map=lambda i: (0, 0)),
    ],
    out_shape=[
        jax.ShapeDtypeStruct((_N,), jnp.int32),
        jax.ShapeDtypeStruct((1, 1), jnp.float32),
    ],
)


@functools.cache
def _make_sc_gather():
    mesh = plsc.VectorSubcoreMesh(core_axis_name="c", subcore_axis_name="s")

    @functools.partial(
        pl.kernel,
        mesh=mesh,
        out_type=jax.ShapeDtypeStruct((_N, _D), jnp.float32),
        scratch_types=[
            pltpu.VMEM((_ROWS_PER_TILE,), jnp.int32),
            pltpu.VMEM((_ROWS_PER_TILE, _D), jnp.float32),
            pltpu.SemaphoreType.DMA,
        ],
        compiler_params=pltpu.CompilerParams(use_tc_tiling_on_sc=False),
    )
    def _sc_gather(cb_hbm, idx_hbm, out_hbm, idx_v, rows_v, sem):
        wid = lax.axis_index("s") * _NC + lax.axis_index("c")
        base = wid * _ROWS_PER_TILE
        pltpu.sync_copy(idx_hbm.at[pl.ds(base, _ROWS_PER_TILE)], idx_v)
        copies = [
            pltpu.async_copy(
                cb_hbm.at[idx_v.at[pl.ds(c * _CHUNK, _CHUNK)]],
                rows_v.at[pl.ds(c * _CHUNK, _CHUNK), :],
                sem,
            )
            for c in range(_NCHUNK)
        ]
        for cp in copies:
            cp.wait()
        pltpu.sync_copy(rows_v, out_hbm.at[pl.ds(base, _ROWS_PER_TILE)])

    return _sc_gather


def kernel(z, codebook):
    B, T, D = z.shape
    flat = z.reshape(_N, D)
    cbn = jnp.sum(codebook ** 2, axis=1)[None, :]
    ids = jnp.arange(_K, dtype=jnp.float32)[None, :]
    idx1, loss = _tc_call(flat, codebook, cbn, ids)
    q = _make_sc_gather()(codebook, idx1)
    return q.reshape(B, T, D), loss.reshape(()), idx1.reshape(B, T)


# speedup vs baseline: 1.0162x; 1.0162x over previous
import functools

import jax
import jax.numpy as jnp
from jax import lax
from jax.experimental import pallas as pl
from jax.experimental.pallas import tpu as pltpu
from jax.experimental.pallas import tpu_sc as plsc

_B, _T, _D = 64, 576, 64
_N = _B * _T
_K = 1024

_R = 1024
_STEPS = _N // _R

_NC, _NS = 2, 16
_NW = _NC * _NS
_ROWS_PER_TILE = _N // _NW
_CHUNK = 128
_NCHUNK = _ROWS_PER_TILE // _CHUNK


def _tc_body(z_ref, cb_ref, cbn_ref, ids_ref, idx_ref, loss_ref):
    i = pl.program_id(0)
    zb = z_ref[...]

    @pl.when(i == 0)
    def _init():
        loss_ref[0, 0] = 0.0

    dots = lax.dot_general(zb, cb_ref[...], (((1,), (1,)), ((), ())),
                           preferred_element_type=jnp.float32)
    rn = jnp.sum(zb * zb, axis=1, keepdims=True)
    d2 = rn - 2.0 * dots + cbn_ref[...]
    m = jnp.min(d2, axis=1)
    ids = jnp.broadcast_to(ids_ref[...], (_R, _K))
    idxf = jnp.min(jnp.where(d2 == m[:, None], ids, float(_K)), axis=1)
    idx_ref[...] = idxf.astype(jnp.int32)
    loss_ref[0, 0] += jnp.sum(m)

    @pl.when(i == _STEPS - 1)
    def _finish():
        loss_ref[0, 0] = loss_ref[0, 0] * (1.25 / (_N * _D))


_tc_call = pl.pallas_call(
    _tc_body,
    grid=(_STEPS,),
    in_specs=[
        pl.BlockSpec((_R, _D), lambda i: (i, 0)),
        pl.BlockSpec((_K, _D), lambda i: (0, 0)),
        pl.BlockSpec((1, _K), lambda i: (0, 0)),
        pl.BlockSpec((1, _K), lambda i: (0, 0)),
    ],
    out_specs=[
        pl.BlockSpec((_R,), lambda i: (i,)),
        pl.BlockSpec(memory_space=pltpu.SMEM, block_shape=(1, 1),
                     index_map=lambda i: (0, 0)),
    ],
    out_shape=[
        jax.ShapeDtypeStruct((_N,), jnp.int32),
        jax.ShapeDtypeStruct((1, 1), jnp.float32),
    ],
)


@functools.cache
def _make_sc_gather():
    mesh = plsc.VectorSubcoreMesh(core_axis_name="c", subcore_axis_name="s")

    @functools.partial(
        pl.kernel,
        mesh=mesh,
        out_type=jax.ShapeDtypeStruct((_N, _D), jnp.float32),
        scratch_types=[
            pltpu.VMEM((_ROWS_PER_TILE,), jnp.int32),
            pltpu.VMEM((_ROWS_PER_TILE, _D), jnp.float32),
            pltpu.SemaphoreType.DMA,
        ],
        compiler_params=pltpu.CompilerParams(use_tc_tiling_on_sc=False),
    )
    def _sc_gather(cb_hbm, idx_hbm, out_hbm, idx_v, rows_v, sem):
        wid = lax.axis_index("s") * _NC + lax.axis_index("c")
        base = wid * _ROWS_PER_TILE
        pltpu.sync_copy(idx_hbm.at[pl.ds(base, _ROWS_PER_TILE)], idx_v)
        copies = [
            pltpu.async_copy(
                cb_hbm.at[idx_v.at[pl.ds(c * _CHUNK, _CHUNK)]],
                rows_v.at[pl.ds(c * _CHUNK, _CHUNK), :],
                sem,
            )
            for c in range(_NCHUNK)
        ]
        for cp in copies:
            cp.wait()
        pltpu.sync_copy(rows_v, out_hbm.at[pl.ds(base, _ROWS_PER_TILE)])

    return _sc_gather


def kernel(z, codebook):
    B, T, D = z.shape
    flat = z.reshape(_N, D)
    cbn = jnp.sum(codebook ** 2, axis=1)[None, :]
    ids = jnp.arange(_K, dtype=jnp.float32)[None, :]
    idx1, loss = _tc_call(flat, codebook, cbn, ids)
    q = _make_sc_gather()(codebook, idx1)
    return q.reshape(B, T, D), loss.reshape(()), idx1.reshape(B, T)
